# Initial kernel scaffold; baseline (speedup 1.0000x reference)
import jax
import jax.numpy as jnp
from jax.experimental import pallas as pl


def kernel(x, Wg, bg, W1, b1, W2, b2):
    raise NotImplementedError("write your pallas kernel here")


# speedup vs baseline: 2.1988x; 2.1988x over previous
import functools

import jax
import jax.numpy as jnp
from jax.experimental import pallas as pl
from jax.experimental.pallas import tpu as pltpu

_BT = 1024


def _gelu_exact(v):
    return 0.5 * v * (1.0 + jax.lax.erf(v * 0.7071067811865476))


def _moe_body(E, xb_ref, wg_ref, bg_ref, w1_ref, b1_ref, w2_ref, b2_ref,
              out_ref, mw_ref):
    e = pl.program_id(1)

    @pl.when(e == 0)
    def _router():
        xb = xb_ref[...]
        logits = jax.lax.dot_general(
            xb, wg_ref[...], (((1,), (1,)), ((), ())),
            preferred_element_type=jnp.float32)
        logits = logits + bg_ref[...]
        iota = jax.lax.broadcasted_iota(jnp.int32, logits.shape, 1)
        m1 = jnp.max(logits, axis=1, keepdims=True)
        idx1 = jnp.min(jnp.where(logits == m1, iota, E), axis=1, keepdims=True)
        masked = jnp.where(iota == idx1, -1e30, logits)
        m2 = jnp.max(masked, axis=1, keepdims=True)
        idx2 = jnp.min(jnp.where(masked == m2, iota, E), axis=1, keepdims=True)
        a = jnp.exp(m2 - m1)
        wt1 = 1.0 / (1.0 + a)
        wt2 = 1.0 - wt1
        mw = jnp.where(iota == idx1, wt1, 0.0) + jnp.where(iota == idx2, wt2, 0.0)
        mw_ref[...] = mw

    xb = xb_ref[...]
    h = jax.lax.dot_general(xb, w1_ref[0], (((1,), (0,)), ((), ())),
                            preferred_element_type=jnp.float32)
    h = _gelu_exact(h + b1_ref[0])
    y = jax.lax.dot_general(h.astype(jnp.bfloat16), w2_ref[0],
                            (((1,), (0,)), ((), ())),
                            preferred_element_type=jnp.float32)
    y = y + b2_ref[0]

    iota = jax.lax.broadcasted_iota(jnp.int32, mw_ref.shape, 1)
    w_e = jnp.sum(jnp.where(iota == e, mw_ref[...], 0.0), axis=1, keepdims=True)

    @pl.when(e == 0)
    def _init():
        out_ref[...] = w_e * y

    @pl.when(e != 0)
    def _acc():
        out_ref[...] += w_e * y


def kernel(x, Wg, bg, W1, b1, W2, b2):
    B_, T_, D_ = x.shape
    E_, H_ = b1.shape
    N = B_ * T_
    NB = N // _BT

    xb = x.reshape(N, D_).astype(jnp.bfloat16)
    wg = Wg.astype(jnp.bfloat16)
    bg2 = bg.reshape(1, E_)
    b1r = b1.reshape(E_, 1, H_)
    b2r = b2.reshape(E_, 1, H_)
    w1t = jnp.swapaxes(W1, 1, 2).astype(jnp.bfloat16)
    w2t = jnp.swapaxes(W2, 1, 2).astype(jnp.bfloat16)

    out = pl.pallas_call(
        functools.partial(_moe_body, E_),
        grid=(NB, E_),
        in_specs=[
            pl.BlockSpec((_BT, D_), lambda i, e: (i, 0)),
            pl.BlockSpec((E_, D_), lambda i, e: (0, 0)),
            pl.BlockSpec((1, E_), lambda i, e: (0, 0)),
            pl.BlockSpec((1, D_, H_), lambda i, e: (e, 0, 0)),
            pl.BlockSpec((1, 1, H_), lambda i, e: (e, 0, 0)),
            pl.BlockSpec((1, H_, H_), lambda i, e: (e, 0, 0)),
            pl.BlockSpec((1, 1, H_), lambda i, e: (e, 0, 0)),
        ],
        out_specs=pl.BlockSpec((_BT, H_), lambda i, e: (i, 0)),
        out_shape=jax.ShapeDtypeStruct((N, H_), jnp.float32),
        scratch_shapes=[pltpu.VMEM((_BT, E_), jnp.float32)],
        compiler_params=pltpu.CompilerParams(
            dimension_semantics=("parallel", "arbitrary")),
    )(xb, wg, bg2, w1t, b1r, w2t, b2r)
    return out.reshape(B_, T_, H_)
